# baseline (speedup 1.0000x reference)
import functools

import jax
import jax.numpy as jnp
from jax import lax
from jax.experimental import pallas as pl
from jax.experimental.pallas import tpu as pltpu
from jax.experimental.pallas import tpu_sc as plsc

VOCAB = 100000
EMBED_DIM = 300
BATCH = 4096
SEQ_LEN = 200
SCALE = 1.0 / SEQ_LEN

NUM_WORKERS = 32
LANES = 16

SC_ROWS = 49152
RPT = SC_ROWS // NUM_WORKERS
CHUNK = 128
NCHUNK = RPT // CHUNK

ROW_BLK = 4096
TC_OFF_BLKS = SC_ROWS // ROW_BLK
TC_BLKS = -(-(VOCAB - SC_ROWS) // ROW_BLK)
TC_ROWS = VOCAB - SC_ROWS


def _matvec_body(e_ref, w_ref, b_ref, o_ref):
    w8 = jnp.broadcast_to(w_ref[...], (8, EMBED_DIM))
    s8 = jax.lax.dot_general(
        w8, e_ref[...],
        dimension_numbers=(((1,), (1,)), ((), ())),
        preferred_element_type=jnp.float32,
    )
    o_ref[...] = s8[0] * SCALE + b_ref[0, 0] * SCALE


def _matvec_tail(e, w, b2):
    return pl.pallas_call(
        _matvec_body,
        grid=(TC_BLKS,),
        in_specs=[
            pl.BlockSpec((ROW_BLK, EMBED_DIM), lambda i: (i + TC_OFF_BLKS, 0)),
            pl.BlockSpec((1, EMBED_DIM), lambda i: (0, 0)),
            pl.BlockSpec((1, 1), lambda i: (0, 0)),
        ],
        out_specs=pl.BlockSpec((ROW_BLK,), lambda i: (i,)),
        out_shape=jax.ShapeDtypeStruct((TC_BLKS * ROW_BLK,), jnp.float32),
    )(e, w, b2)


def _sc_mesh():
    return plsc.VectorSubcoreMesh(core_axis_name="c", subcore_axis_name="s")


@functools.cache
def _make_sc_matvec():
    @functools.partial(
        pl.kernel,
        mesh=_sc_mesh(),
        out_type=jax.ShapeDtypeStruct((SC_ROWS,), jnp.float32),
        scratch_types=[
            pltpu.VMEM((CHUNK * EMBED_DIM,), jnp.float32),
            pltpu.VMEM((CHUNK * EMBED_DIM,), jnp.float32),
            pltpu.VMEM((EMBED_DIM * LANES,), jnp.float32),
            pltpu.VMEM((LANES,), jnp.float32),
            pltpu.VMEM((RPT,), jnp.float32),
            pltpu.SemaphoreType.DMA,
            pltpu.SemaphoreType.DMA,
            pltpu.SemaphoreType.DMA,
            pltpu.SemaphoreType.DMA,
        ],
        compiler_params=pltpu.CompilerParams(
            use_tc_tiling_on_sc=False, needs_layout_passes=False),
    )
    def _sc_matvec(ef_hbm, w_hbm, b_hbm, s_hbm, e0, e1, w_v, b_v, out_v,
                   sem0, sem1, sem_w, sem_b):
        wid = lax.axis_index("s") * 2 + lax.axis_index("c")
        base_f = wid * (RPT * EMBED_DIM)
        cp_w = pltpu.make_async_copy(w_hbm, w_v, sem_w)
        cp_b = pltpu.make_async_copy(b_hbm, b_v, sem_b)
        cp_w.start()
        cp_b.start()
        bufs = (e0, e1)
        sems = (sem0, sem1)
        cwords = CHUNK * EMBED_DIM
        cp = pltpu.make_async_copy(ef_hbm.at[pl.ds(base_f, cwords)], e0, sem0)
        cp.start()
        cp_w.wait()
        cp_b.wait()
        bias = b_v[...] * SCALE
        for c in range(NCHUNK):
            nxt = None
            if c + 1 < NCHUNK:
                nxt = pltpu.make_async_copy(
                    ef_hbm.at[pl.ds(base_f + (c + 1) * cwords, cwords)],
                    bufs[(c + 1) % 2], sems[(c + 1) % 2])
                nxt.start()
            cp.wait()
            e_v = bufs[c % 2]
            for sg in range(2):
                r16 = tuple(
                    (lax.iota(jnp.int32, LANES) + (sg * 64 + g * LANES))
                    * EMBED_DIM
                    for g in range(4))

                def body(l, accs):
                    wl = w_v[pl.ds(l * LANES, LANES)]
                    return tuple(
                        accs[g]
                        + plsc.load_gather(e_v, [r16[g] + l]) * wl
                        for g in range(4))

                accs = lax.fori_loop(
                    0, EMBED_DIM, body,
                    tuple(jnp.zeros((LANES,), jnp.float32) for _ in range(4)))
                for g in range(4):
                    out_v[pl.ds(c * CHUNK + sg * 64 + g * LANES, LANES)] = (
                        accs[g] * SCALE + bias)
            cp = nxt
        pltpu.sync_copy(out_v, s_hbm.at[pl.ds(wid * RPT, RPT)])

    return _sc_matvec


BPW = BATCH // NUM_WORKERS
GROUPS = BPW // LANES


@functools.cache
def _make_sc_bag():
    @functools.partial(
        pl.kernel,
        mesh=_sc_mesh(),
        out_type=jax.ShapeDtypeStruct((BATCH,), jnp.float32),
        scratch_types=[
            pltpu.VMEM((VOCAB,), jnp.float32),
            pltpu.VMEM_SHARED((VOCAB,), jnp.float32),
            pltpu.VMEM((BPW // 2, SEQ_LEN), jnp.int32),
            pltpu.VMEM((BPW,), jnp.float32),
            pltpu.SemaphoreType.DMA,
            pltpu.SemaphoreType.DMA,
        ],
        compiler_params=pltpu.CompilerParams(
            use_tc_tiling_on_sc=False, needs_layout_passes=False),
    )
    def _sc_bag(s_hi_hbm, s_lo_hbm, idx_hbm, out_hbm, s_v, s_sh, idx_v,
                out_v, sem_s, sem_i):
        num_cores = 2
        sid = lax.axis_index("s")
        wid = sid * num_cores + lax.axis_index("c")
        base = wid * BPW
        cp_i = pltpu.make_async_copy(
            idx_hbm.at[pl.ds(base, BPW // 2), :], idx_v, sem_i)
        cp_i.start()
        @pl.when(sid == 0)
        def _():
            pltpu.sync_copy(s_hi_hbm, s_sh.at[pl.ds(0, SC_ROWS)])
            pltpu.sync_copy(
                s_lo_hbm.at[pl.ds(0, TC_ROWS)],
                s_sh.at[pl.ds(SC_ROWS, TC_ROWS)])
        plsc.subcore_barrier()
        cp_s = pltpu.make_async_copy(s_sh, s_v, sem_s)
        cp_s.start()
        cp_s.wait()
        cp_i.wait()

        half_groups = GROUPS // 2
        for half in range(2):
            for kk in range(half_groups):
                k = half * half_groups + kk
                rows = lax.iota(jnp.int32, LANES) + (kk * LANES)

                def body(l, acc):
                    cols = jnp.full((LANES,), l, jnp.int32)
                    iv = plsc.load_gather(idx_v, [rows, cols])
                    vals = plsc.load_gather(s_v, [iv])
                    return acc + vals

                acc = lax.fori_loop(
                    0, SEQ_LEN, body, jnp.zeros((LANES,), jnp.float32))
                out_v[pl.ds(k * LANES, LANES)] = acc
            if half == 0:
                pltpu.sync_copy(
                    idx_hbm.at[pl.ds(base + BPW // 2, BPW // 2), :], idx_v)

        pltpu.sync_copy(out_v, out_hbm.at[pl.ds(base, BPW)])

    return _sc_bag


@jax.jit
def kernel(input_words, embedding, fc1_w, fc1_b):
    b2 = fc1_b.reshape(1, 1)
    e_flat = embedding.reshape(VOCAB * EMBED_DIM)
    w_rep = jnp.repeat(fc1_w.reshape(EMBED_DIM), LANES)
    b16 = jnp.broadcast_to(fc1_b, (LANES,))
    s_hi = _make_sc_matvec()(e_flat, w_rep, b16)
    s_lo = _matvec_tail(embedding, fc1_w, b2)
    return _make_sc_bag()(s_hi, s_lo, input_words)
